# Initial kernel scaffold; baseline (speedup 1.0000x reference)
import jax
import jax.numpy as jnp
from jax.experimental import pallas as pl


def kernel(p_coords, p_features, v_indices, v_features, v_map, v_mask, W, gamma, beta):
    raise NotImplementedError("write your pallas kernel here")


# speedup vs baseline: 2.4304x; 2.4304x over previous
import functools

import jax
import jax.numpy as jnp
from jax import lax
from jax.experimental import pallas as pl
from jax.experimental.pallas import tpu as pltpu
from jax.experimental.pallas import tpu_sc as plsc

N = 100000
M = 35000
NS = 16
C = 128
D = 144
VX, VY, VZ = 0.1, 0.1, 0.15
X0, Y0, Z0 = 0.0, -40.0, -3.0
KN = VX * VX + VY * VY + VZ * VZ

NC, NSC = 2, 16
NW = NC * NSC

NPAD = 100096
SLC = NPAD // NSC
VTOT = M * NS
VTOT_PAD = 573440
QW = VTOT_PAD // NW
NCHUNK_A = QW // 128

CH = 8
ROWS = CH * NS
WCH = 137
MPAD = NW * WCH * CH


def _counts_body(vmap_hbm, out_hbm, idx_v, ones_v, zbuf_v, csh):
    cid = lax.axis_index("c")
    sid = lax.axis_index("s")
    wid = sid * NC + cid

    ones16 = jnp.full((16,), 1.0, jnp.float32)
    for b in range(8):
        ones_v[pl.ds(b * 16, 16)] = ones16

    z16 = jnp.zeros((16,), jnp.float32)

    @pl.loop(0, SLC // 16)
    def _zero(i):
        zbuf_v[pl.ds(i * 16, 16)] = z16

    pltpu.sync_copy(zbuf_v, csh.at[pl.ds(sid * SLC, SLC)])
    plsc.subcore_barrier()

    base = wid * QW

    @pl.loop(0, NCHUNK_A)
    def _chunk(k):
        pltpu.sync_copy(vmap_hbm.at[pl.ds(base + k * 128, 128)], idx_v)
        pltpu.sync_copy(ones_v, csh.at[idx_v], add=True)

    plsc.subcore_barrier()
    pltpu.sync_copy(csh.at[pl.ds(sid * SLC, SLC)], zbuf_v)
    pltpu.sync_copy(zbuf_v, out_hbm.at[pl.ds(cid * NPAD + sid * SLC, SLC)])


@jax.jit
def _counts_call(vmap_pad):
    mesh = plsc.VectorSubcoreMesh(core_axis_name="c", subcore_axis_name="s",
                                  num_cores=NC, num_subcores=NSC)
    return pl.kernel(
        _counts_body,
        out_type=jax.ShapeDtypeStruct((NC * NPAD,), jnp.float32),
        mesh=mesh,
        scratch_types=[
            pltpu.VMEM((128,), jnp.int32),
            pltpu.VMEM((128,), jnp.float32),
            pltpu.VMEM((SLC,), jnp.float32),
            pltpu.VMEM_SHARED((NPAD,), jnp.float32),
        ],
    )(vmap_pad)


BN = 1000
NB = N // BN


def _stats_body(cnt_ref, pf_ref, wt_ref, out_ref):
    i = pl.program_id(0)

    @pl.when(i == 0)
    def _():
        out_ref[...] = jnp.zeros_like(out_ref)

    h = jnp.dot(pf_ref[...], wt_ref[...], preferred_element_type=jnp.float32)
    cw = cnt_ref[:, 0:1] + cnt_ref[:, 1:2]
    ch = cw * h
    out_ref[0:1, :] += jnp.sum(ch, axis=0, keepdims=True)
    out_ref[1:2, :] += jnp.sum(ch * h, axis=0, keepdims=True)


@jax.jit
def _stats_call(counts_nx2, p_features, wt):
    return pl.pallas_call(
        _stats_body,
        grid=(NB,),
        in_specs=[
            pl.BlockSpec((BN, 2), lambda i: (i, 0)),
            pl.BlockSpec((BN, C), lambda i: (i, 0)),
            pl.BlockSpec((C, C), lambda i: (0, 0)),
        ],
        out_specs=pl.BlockSpec((2, C), lambda i: (0, 0)),
        out_shape=jax.ShapeDtypeStruct((2, C), jnp.float32),
    )(counts_nx2, p_features, wt)


def _apply_body(pf_ref, pc_ref, wt_ref, t8_ref, ab_ref, out_ref):
    h = jnp.dot(pf_ref[...], wt_ref[...], preferred_element_type=jnp.float32)
    hpp = jnp.maximum(h * ab_ref[0:1, :] + ab_ref[1:2, :], 0.0)
    pc = pc_ref[...]
    ext = jnp.concatenate([pc, pc * pc], axis=1)
    q16 = jnp.dot(ext, t8_ref[...], preferred_element_type=jnp.float32)
    out_ref[...] = jnp.concatenate([hpp, q16], axis=1)


@jax.jit
def _apply_call(p_features, p_coords, wt, t8, ab):
    return pl.pallas_call(
        _apply_body,
        grid=(NB,),
        in_specs=[
            pl.BlockSpec((BN, C), lambda i: (i, 0)),
            pl.BlockSpec((BN, 4), lambda i: (i, 0)),
            pl.BlockSpec((C, C), lambda i: (0, 0)),
            pl.BlockSpec((8, 16), lambda i: (0, 0)),
            pl.BlockSpec((2, C), lambda i: (0, 0)),
        ],
        out_specs=pl.BlockSpec((BN, D), lambda i: (i, 0)),
        out_shape=jax.ShapeDtypeStruct((N, D), jnp.float32),
    )(p_features, p_coords, wt, t8, ab)


BM = 1096
NBM = MPAD // BM


def _vox_body(vi_ref, vm_ref, vp_ref, rmd_ref):
    vi = vi_ref[...].astype(jnp.float32)
    cx = (vi[:, 3:4] + 0.5) * VX + X0
    cy = (vi[:, 2:3] + 0.5) * VY + Y0
    cz = (vi[:, 1:2] + 0.5) * VZ + Z0
    ux = cx * (1.0 / VX)
    uy = cy * (1.0 / VY)
    uz = cz * (1.0 / VZ)
    ud = (cx * cx + cy * cy + cz * cz) * (1.0 / KN)
    tx = cx * (2.0 * VX / KN)
    ty = cy * (2.0 * VY / KN)
    tz = cz * (2.0 * VZ / KN)
    scal8 = jnp.concatenate([ux, uy, uz, ud, tx, ty, tz,
                             jnp.zeros_like(ux)], axis=1)
    vp_ref[...] = jnp.broadcast_to(scal8[:, :, None], (BM, 8, 16))
    vm = vm_ref[...]
    s = jnp.sum(vm, axis=1, keepdims=True)
    rmd_ref[...] = vm / jnp.maximum(s, 1.0)


@jax.jit
def _vox_call(vi_pad, vm_pad):
    return pl.pallas_call(
        _vox_body,
        grid=(NBM,),
        in_specs=[
            pl.BlockSpec((BM, 4), lambda i: (i, 0)),
            pl.BlockSpec((BM, 16), lambda i: (i, 0)),
        ],
        out_specs=[
            pl.BlockSpec((BM, 8, 16), lambda i: (i, 0, 0)),
            pl.BlockSpec((BM, 16), lambda i: (i, 0)),
        ],
        out_shape=[
            jax.ShapeDtypeStruct((MPAD, 8, 16), jnp.float32),
            jax.ShapeDtypeStruct((MPAD, 16), jnp.float32),
        ],
    )(vi_pad, vm_pad)


def _pool_body(hpq_hbm, vmapf_hbm, vp_hbm, rmd_hbm, vf_hbm, out_hbm,
               idx_v, rows_v, vp_v, rmd_v, vf_v, out_v, wbuf_v, gsem):
    cid = lax.axis_index("c")
    sid = lax.axis_index("s")
    wid = sid * NC + cid
    cbase = wid * WCH

    iota = lax.iota(jnp.int32, 16)
    c128 = iota * 0 + 128
    c129 = iota * 0 + 129
    c130 = iota * 0 + 130
    c131 = iota * 0 + 131
    widx0 = (iota % 4) * 16

    @pl.loop(0, WCH)
    def _chunk(k):
        vb = (cbase + k) * CH
        pltpu.sync_copy(vmapf_hbm.at[pl.ds(vb * NS, ROWS)], idx_v)
        pltpu.async_copy(hpq_hbm.at[idx_v], rows_v, gsem).wait()
        pltpu.sync_copy(vp_hbm.at[pl.ds(vb, CH)], vp_v)
        pltpu.sync_copy(rmd_hbm.at[pl.ds(vb, CH)], rmd_v)
        pltpu.sync_copy(vf_hbm.at[pl.ds(vb, CH)], vf_v)

        @pl.loop(0, CH)
        def _vox(i):
            r0 = i * NS
            ridx = r0 + iota
            qx = plsc.load_gather(rows_v, [ridx, c128])
            qy = plsc.load_gather(rows_v, [ridx, c129])
            qz = plsc.load_gather(rows_v, [ridx, c130])
            qd = plsc.load_gather(rows_v, [ridx, c131])
            ux = vp_v[i, 0, :]
            uy = vp_v[i, 1, :]
            uz = vp_v[i, 2, :]
            ud = vp_v[i, 3, :]
            tx = vp_v[i, 4, :]
            ty = vp_v[i, 5, :]
            tz = vp_v[i, 6, :]
            mds = rmd_v[i, :]
            g0 = qx - ux
            g1 = qy - uy
            g2 = qz - uz
            g3 = qd + ud - tx * qx - ty * qy - tz * qz
            wbuf_v[pl.ds(0, 16)] = g0 * mds
            wbuf_v[pl.ds(16, 16)] = g1 * mds
            wbuf_v[pl.ds(32, 16)] = g2 * mds
            wbuf_v[pl.ds(48, 16)] = g3 * mds
            acc = [vf_v[i, pl.ds(j * 16, 16)] for j in range(8)]
            for s in range(NS):
                wv = plsc.load_gather(wbuf_v, [widx0 + s])
                r = r0 + s
                for j in range(8):
                    acc[j] = acc[j] + wv * rows_v[r, pl.ds(j * 16, 16)]
            for j in range(8):
                out_v[i, pl.ds(j * 16, 16)] = acc[j]

        pltpu.sync_copy(out_v, out_hbm.at[pl.ds(vb, CH)])


@jax.jit
def _pool_call(hpq, vmapf, vp, rmd, vf):
    mesh = plsc.VectorSubcoreMesh(core_axis_name="c", subcore_axis_name="s",
                                  num_cores=NC, num_subcores=NSC)
    return pl.kernel(
        _pool_body,
        out_type=jax.ShapeDtypeStruct((MPAD, C), jnp.float32),
        mesh=mesh,
        compiler_params=pltpu.CompilerParams(use_tc_tiling_on_sc=False,
                                             needs_layout_passes=False),
        scratch_types=[
            pltpu.VMEM((ROWS,), jnp.int32),
            pltpu.VMEM((ROWS, D), jnp.float32),
            pltpu.VMEM((CH, 8, 16), jnp.float32),
            pltpu.VMEM((CH, 16), jnp.float32),
            pltpu.VMEM((CH, C), jnp.float32),
            pltpu.VMEM((CH, C), jnp.float32),
            pltpu.VMEM((64,), jnp.float32),
            pltpu.SemaphoreType.DMA,
        ],
    )(hpq, vmapf, vp, rmd, vf)


def kernel(p_coords, p_features, v_indices, v_features, v_map, v_mask, W,
           gamma, beta):
    vmap_flat = v_map.astype(jnp.int32).reshape(-1)
    vmap_a = jnp.concatenate(
        [vmap_flat, jnp.full((VTOT_PAD - VTOT,), N, jnp.int32)])
    counts_flat = _counts_call(vmap_a)
    counts_nx2 = jnp.stack([counts_flat[:N], counts_flat[NPAD:NPAD + N]],
                           axis=1)

    wt = W.T
    stats = _stats_call(counts_nx2, p_features, wt)
    rtot = float(M * NS)
    mean = stats[0] / rtot
    var = stats[1] / rtot - mean * mean
    a = gamma * lax.rsqrt(var + 1e-5)
    b = beta - mean * a
    ab = jnp.stack([a, b])

    t8 = jnp.zeros((8, 16), jnp.float32)
    t8 = t8.at[1, 0].set(1.0 / VX)
    t8 = t8.at[2, 1].set(1.0 / VY)
    t8 = t8.at[3, 2].set(1.0 / VZ)
    t8 = t8.at[5, 3].set(1.0 / KN)
    t8 = t8.at[6, 3].set(1.0 / KN)
    t8 = t8.at[7, 3].set(1.0 / KN)
    hpq = _apply_call(p_features, p_coords, wt, t8, ab)

    vi_pad = jnp.pad(v_indices.astype(jnp.int32), ((0, MPAD - M), (0, 0)))
    vm_pad = jnp.pad(v_mask, ((0, MPAD - M), (0, 0)))
    vp, rmd = _vox_call(vi_pad, vm_pad)

    vf_pad = jnp.pad(v_features, ((0, MPAD - M), (0, 0)))
    vmap_c = jnp.pad(vmap_flat, (0, (MPAD - M) * NS))

    out = _pool_call(hpq, vmap_c, vp, rmd, vf_pad)
    return out[:M]


# speedup vs baseline: 3.5181x; 1.4475x over previous
import jax
import jax.numpy as jnp
from jax import lax
from jax.experimental import pallas as pl
from jax.experimental.pallas import tpu as pltpu
from jax.experimental.pallas import tpu_sc as plsc

N = 100000
M = 35000
NS = 16
C = 128
D = 144
VX, VY, VZ = 0.1, 0.1, 0.15
X0, Y0, Z0 = 0.0, -40.0, -3.0
KN = VX * VX + VY * VY + VZ * VZ
S0, S1, S2 = VX * VX / KN, VY * VY / KN, VZ * VZ / KN

NC, NSC = 2, 16
NW = NC * NSC

NPAD = 100096
SLC = NPAD // NSC
VTOT = M * NS
RND_A = 18
QW = RND_A * 8 * 128
VTOT_PAD = NW * QW

CH = 8
ROWSN = CH * NS
WCH = 138
MPAD = NW * WCH * CH


def _counts_body(vmap_hbm, out_hbm, idx_a, idx_b, ones_v, zbuf_v, csh,
                 isem_a, isem_b, asem_a, asem_b):
    cid = lax.axis_index("c")
    sid = lax.axis_index("s")
    wid = sid * NC + cid
    base = wid * QW

    ones16 = jnp.full((16,), 1.0, jnp.float32)
    for b in range(8):
        ones_v[pl.ds(b * 16, 16)] = ones16

    z16 = jnp.zeros((16,), jnp.float32)

    @pl.loop(0, SLC // 16)
    def _zero(i):
        zbuf_v[pl.ds(i * 16, 16)] = z16

    pltpu.sync_copy(zbuf_v, csh.at[pl.ds(sid * SLC, SLC)])
    plsc.subcore_barrier()

    def issue_idx(idx_ref, sem, r):
        for b in range(8):
            pltpu.async_copy(
                vmap_hbm.at[pl.ds(base + (r * 8 + b) * 128, 128)],
                idx_ref.at[b], sem)

    def drain_idx(idx_ref, sem):
        for b in range(8):
            pltpu.make_async_copy(vmap_hbm.at[pl.ds(base, 128)],
                                  idx_ref.at[b], sem).wait()

    def issue_add(idx_ref, sem):
        for b in range(8):
            pltpu.async_copy(ones_v, csh.at[idx_ref.at[b]], sem, add=True)

    def drain_add(idx_ref, sem):
        for b in range(8):
            pltpu.make_async_copy(ones_v, csh.at[idx_ref.at[b]], sem).wait()

    issue_idx(idx_a, isem_a, 0)

    @pl.loop(0, RND_A // 2)
    def _t(t):
        @pl.when(t > 0)
        def _():
            drain_add(idx_b, asem_b)
        issue_idx(idx_b, isem_b, 2 * t + 1)
        drain_idx(idx_a, isem_a)
        issue_add(idx_a, asem_a)
        drain_add(idx_a, asem_a)

        @pl.when(t < RND_A // 2 - 1)
        def _():
            issue_idx(idx_a, isem_a, 2 * t + 2)
        drain_idx(idx_b, isem_b)
        issue_add(idx_b, asem_b)

    drain_add(idx_b, asem_b)
    plsc.subcore_barrier()
    pltpu.sync_copy(csh.at[pl.ds(sid * SLC, SLC)], zbuf_v)
    pltpu.sync_copy(zbuf_v, out_hbm.at[pl.ds(cid * NPAD + sid * SLC, SLC)])


@jax.jit
def _counts_call(vmap_pad):
    mesh = plsc.VectorSubcoreMesh(core_axis_name="c", subcore_axis_name="s",
                                  num_cores=NC, num_subcores=NSC)
    return pl.kernel(
        _counts_body,
        out_type=jax.ShapeDtypeStruct((NC * NPAD,), jnp.float32),
        mesh=mesh,
        scratch_types=[
            pltpu.VMEM((8, 128), jnp.int32),
            pltpu.VMEM((8, 128), jnp.int32),
            pltpu.VMEM((128,), jnp.float32),
            pltpu.VMEM((SLC,), jnp.float32),
            pltpu.VMEM_SHARED((NPAD,), jnp.float32),
            pltpu.SemaphoreType.DMA,
            pltpu.SemaphoreType.DMA,
            pltpu.SemaphoreType.DMA,
            pltpu.SemaphoreType.DMA,
        ],
    )(vmap_pad)


BN = 2000
NB = N // BN


def _stats_body(cnt_ref, pf_ref, wt_ref, out_ref):
    i = pl.program_id(0)

    @pl.when(i == 0)
    def _():
        out_ref[...] = jnp.zeros_like(out_ref)

    h = jnp.dot(pf_ref[...], wt_ref[...], preferred_element_type=jnp.float32)
    cw = cnt_ref[:, 0:1] + cnt_ref[:, 1:2]
    ch = cw * h
    out_ref[0:1, :] += jnp.sum(ch, axis=0, keepdims=True)
    out_ref[1:2, :] += jnp.sum(ch * h, axis=0, keepdims=True)


@jax.jit
def _stats_call(counts_nx2, p_features, wt):
    return pl.pallas_call(
        _stats_body,
        grid=(NB,),
        in_specs=[
            pl.BlockSpec((BN, 2), lambda i: (i, 0)),
            pl.BlockSpec((BN, C), lambda i: (i, 0)),
            pl.BlockSpec((C, C), lambda i: (0, 0)),
        ],
        out_specs=pl.BlockSpec((2, C), lambda i: (0, 0)),
        out_shape=jax.ShapeDtypeStruct((2, C), jnp.float32),
    )(counts_nx2, p_features, wt)


def _apply_body(pf_ref, pc_ref, wt_ref, t8_ref, ab_ref, out_ref):
    h = jnp.dot(pf_ref[...], wt_ref[...], preferred_element_type=jnp.float32)
    hpp = jnp.maximum(h * ab_ref[0:1, :] + ab_ref[1:2, :], 0.0)
    pc = pc_ref[...]
    q16 = jnp.dot(pc, t8_ref[...], preferred_element_type=jnp.float32)
    out_ref[...] = jnp.concatenate([hpp, q16], axis=1)


@jax.jit
def _apply_call(p_features, p_coords, wt, t8, ab):
    return pl.pallas_call(
        _apply_body,
        grid=(NB,),
        in_specs=[
            pl.BlockSpec((BN, C), lambda i: (i, 0)),
            pl.BlockSpec((BN, 4), lambda i: (i, 0)),
            pl.BlockSpec((C, C), lambda i: (0, 0)),
            pl.BlockSpec((4, 16), lambda i: (0, 0)),
            pl.BlockSpec((2, C), lambda i: (0, 0)),
        ],
        out_specs=pl.BlockSpec((BN, D), lambda i: (i, 0)),
        out_shape=jax.ShapeDtypeStruct((N, D), jnp.float32),
    )(p_features, p_coords, wt, t8, ab)


BM = 1104
NBM = MPAD // BM


def _vox_body(vi_ref, vm_ref, vf_ref, aux_ref):
    vi = vi_ref[...].astype(jnp.float32)
    ux = ((vi[:, 3:4] + 0.5) * VX + X0) * (1.0 / VX)
    uy = ((vi[:, 2:3] + 0.5) * VY + Y0) * (1.0 / VY)
    uz = ((vi[:, 1:2] + 0.5) * VZ + Z0) * (1.0 / VZ)
    scal3 = jnp.concatenate([ux, uy, uz], axis=1)
    bc = jnp.broadcast_to(scal3[:, :, None], (BM, 3, 16)).reshape(BM, 48)
    vm = vm_ref[...]
    s = jnp.sum(vm, axis=1, keepdims=True)
    rmd = vm / jnp.maximum(s, 1.0)
    plane0 = jnp.concatenate([bc, rmd, jnp.zeros((BM, 64), jnp.float32)],
                             axis=1)
    aux_ref[...] = jnp.stack([plane0, vf_ref[...]], axis=1)


@jax.jit
def _vox_call(vi_pad, vm_pad, vf_pad):
    return pl.pallas_call(
        _vox_body,
        grid=(NBM,),
        in_specs=[
            pl.BlockSpec((BM, 4), lambda i: (i, 0)),
            pl.BlockSpec((BM, 16), lambda i: (i, 0)),
            pl.BlockSpec((BM, C), lambda i: (i, 0)),
        ],
        out_specs=pl.BlockSpec((BM, 2, C), lambda i: (i, 0, 0)),
        out_shape=jax.ShapeDtypeStruct((MPAD, 2, C), jnp.float32),
    )(vi_pad, vm_pad, vf_pad)


def _pool_body(hpq_hbm, vmapf_hbm, aux_hbm, out_hbm,
               idx0, idx1, rows0, rows1, aux0, aux1, out0, out1, wbuf,
               isem0, isem1, gsem0, gsem1, xsem0, xsem1, osem0, osem1):
    cid = lax.axis_index("c")
    sid = lax.axis_index("s")
    wid = sid * NC + cid
    cbase = wid * WCH

    IDX = (idx0, idx1)
    ROWS = (rows0, rows1)
    AUX = (aux0, aux1)
    OUT = (out0, out1)
    ISEM = (isem0, isem1)
    GSEM = (gsem0, gsem1)
    XSEM = (xsem0, xsem1)
    OSEM = (osem0, osem1)

    iota = lax.iota(jnp.int32, 16)
    c128 = iota * 0 + 128
    c129 = iota * 0 + 129
    c130 = iota * 0 + 130
    widx = [(iota % 4) * 16 + s for s in range(NS)]

    def vbase(k):
        return (cbase + k) * CH

    def issue_idx(p, k):
        pltpu.async_copy(vmapf_hbm.at[pl.ds(vbase(k) * NS, ROWSN)],
                         IDX[p], ISEM[p])

    def wait_idx(p):
        pltpu.make_async_copy(vmapf_hbm.at[pl.ds(0, ROWSN)],
                              IDX[p], ISEM[p]).wait()

    def issue_gather(p):
        pltpu.async_copy(hpq_hbm.at[IDX[p]], ROWS[p], GSEM[p])

    def wait_gather(p):
        pltpu.make_async_copy(hpq_hbm.at[IDX[p]], ROWS[p], GSEM[p]).wait()

    def issue_aux(p, k):
        pltpu.async_copy(aux_hbm.at[pl.ds(vbase(k), CH)], AUX[p], XSEM[p])

    def wait_aux(p):
        pltpu.make_async_copy(aux_hbm.at[pl.ds(0, CH)], AUX[p],
                              XSEM[p]).wait()

    def issue_out(p, k):
        pltpu.async_copy(OUT[p], out_hbm.at[pl.ds(vbase(k), CH)], OSEM[p])

    def wait_out(p):
        pltpu.make_async_copy(OUT[p], out_hbm.at[pl.ds(0, CH)],
                              OSEM[p]).wait()

    def compute(p):
        rows_v = ROWS[p]
        aux_v = AUX[p]
        out_v = OUT[p]

        @pl.loop(0, CH)
        def _vox(i):
            ridx = i * NS + iota
            qx = plsc.load_gather(rows_v, [ridx, c128])
            qy = plsc.load_gather(rows_v, [ridx, c129])
            qz = plsc.load_gather(rows_v, [ridx, c130])
            ux = aux_v[i, 0, pl.ds(0, 16)]
            uy = aux_v[i, 0, pl.ds(16, 16)]
            uz = aux_v[i, 0, pl.ds(32, 16)]
            mds = aux_v[i, 0, pl.ds(48, 16)]
            g0 = qx - ux
            g1 = qy - uy
            g2 = qz - uz
            g3 = S0 * (g0 * g0) + S1 * (g1 * g1) + S2 * (g2 * g2)
            wbuf[pl.ds(0, 16)] = g0 * mds
            wbuf[pl.ds(16, 16)] = g1 * mds
            wbuf[pl.ds(32, 16)] = g2 * mds
            wbuf[pl.ds(48, 16)] = g3 * mds
            acc = [aux_v[i, 1, pl.ds(j * 16, 16)] for j in range(8)]
            for s in range(NS):
                wv = plsc.load_gather(wbuf, [widx[s]])
                r = i * NS + s
                for j in range(8):
                    acc[j] = acc[j] + wv * rows_v[r, pl.ds(j * 16, 16)]
            for j in range(8):
                out_v[i, pl.ds(j * 16, 16)] = acc[j]

    issue_idx(0, 0)
    wait_idx(0)
    issue_gather(0)
    issue_aux(0, 0)
    issue_idx(1, 1)

    @pl.loop(0, WCH // 2)
    def _t(t):
        for p in (0, 1):
            k = 2 * t + p

            @pl.when(k + 1 < WCH)
            def _():
                wait_idx(1 - p)
                issue_gather(1 - p)
                issue_aux(1 - p, k + 1)
            wait_gather(p)
            wait_aux(p)

            @pl.when(k + 2 < WCH)
            def _():
                issue_idx(p, k + 2)

            @pl.when(k >= 2)
            def _():
                wait_out(p)
            compute(p)
            issue_out(p, k)

    wait_out(0)
    wait_out(1)


@jax.jit
def _pool_call(hpq, vmapf, aux):
    mesh = plsc.VectorSubcoreMesh(core_axis_name="c", subcore_axis_name="s",
                                  num_cores=NC, num_subcores=NSC)
    return pl.kernel(
        _pool_body,
        out_type=jax.ShapeDtypeStruct((MPAD, C), jnp.float32),
        mesh=mesh,
        compiler_params=pltpu.CompilerParams(use_tc_tiling_on_sc=False,
                                             needs_layout_passes=False),
        scratch_types=[
            pltpu.VMEM((ROWSN,), jnp.int32),
            pltpu.VMEM((ROWSN,), jnp.int32),
            pltpu.VMEM((ROWSN, D), jnp.float32),
            pltpu.VMEM((ROWSN, D), jnp.float32),
            pltpu.VMEM((CH, 2, C), jnp.float32),
            pltpu.VMEM((CH, 2, C), jnp.float32),
            pltpu.VMEM((CH, C), jnp.float32),
            pltpu.VMEM((CH, C), jnp.float32),
            pltpu.VMEM((64,), jnp.float32),
            pltpu.SemaphoreType.DMA,
            pltpu.SemaphoreType.DMA,
            pltpu.SemaphoreType.DMA,
            pltpu.SemaphoreType.DMA,
            pltpu.SemaphoreType.DMA,
            pltpu.SemaphoreType.DMA,
            pltpu.SemaphoreType.DMA,
            pltpu.SemaphoreType.DMA,
        ],
    )(hpq, vmapf, aux)


def kernel(p_coords, p_features, v_indices, v_features, v_map, v_mask, W,
           gamma, beta):
    vmap_flat = v_map.astype(jnp.int32).reshape(-1)
    vmap_a = jnp.concatenate(
        [vmap_flat, jnp.full((VTOT_PAD - VTOT,), N, jnp.int32)])
    counts_flat = _counts_call(vmap_a)
    counts_nx2 = jnp.stack([counts_flat[:N], counts_flat[NPAD:NPAD + N]],
                           axis=1)

    wt = W.T
    stats = _stats_call(counts_nx2, p_features, wt)
    rtot = float(M * NS)
    mean = stats[0] / rtot
    var = stats[1] / rtot - mean * mean
    a = gamma * lax.rsqrt(var + 1e-5)
    b = beta - mean * a
    ab = jnp.stack([a, b])

    t8 = jnp.zeros((4, 16), jnp.float32)
    t8 = t8.at[1, 0].set(1.0 / VX)
    t8 = t8.at[2, 1].set(1.0 / VY)
    t8 = t8.at[3, 2].set(1.0 / VZ)
    hpq = _apply_call(p_features, p_coords, wt, t8, ab)

    vi_pad = jnp.pad(v_indices.astype(jnp.int32), ((0, MPAD - M), (0, 0)))
    vm_pad = jnp.pad(v_mask, ((0, MPAD - M), (0, 0)))
    vf_pad = jnp.pad(v_features, ((0, MPAD - M), (0, 0)))
    aux = _vox_call(vi_pad, vm_pad, vf_pad)

    vmap_c = jnp.pad(vmap_flat, (0, (MPAD - M) * NS))

    out = _pool_call(hpq, vmap_c, aux)
    return out[:M]


# speedup vs baseline: 3.7976x; 1.0795x over previous
import jax
import jax.numpy as jnp
from jax import lax
from jax.experimental import pallas as pl
from jax.experimental.pallas import tpu as pltpu
from jax.experimental.pallas import tpu_sc as plsc

N = 100000
M = 35000
NS = 16
C = 128
VX, VY, VZ = 0.1, 0.1, 0.15
X0, Y0, Z0 = 0.0, -40.0, -3.0
KN = VX * VX + VY * VY + VZ * VZ
S0, S1, S2 = VX * VX / KN, VY * VY / KN, VZ * VZ / KN
CX0, CY0, CZ0 = 0.5 + X0 / VX, 0.5 + Y0 / VY, 0.5 + Z0 / VZ

NC, NSC = 2, 16
NW = NC * NSC

NPAD = 100096
SLC = NPAD // NSC
VTOT = M * NS
RND_A = 18
QW = RND_A * 8 * 128
VTOT_PAD = NW * QW
NQCH = 26
NQW = NQCH * 128
QTPAD = NW * NQW

CH = 8
ROWSN = CH * NS
WCH = 138
MPAD = NW * WCH * CH

_SC_PARAMS = pltpu.CompilerParams(use_tc_tiling_on_sc=False,
                                  needs_layout_passes=False)


def _sc_mesh():
    return plsc.VectorSubcoreMesh(core_axis_name="c", subcore_axis_name="s",
                                  num_cores=NC, num_subcores=NSC)


def _counts_body(vmap_hbm, pc_hbm, out_hbm, qtab_hbm,
                 idx_a, idx_b, pc0, pc1, qb0, qb1, ones_v, zbuf_v, csh,
                 isem_a, isem_b, asem_a, asem_b,
                 psem0, psem1, qsem0, qsem1):
    cid = lax.axis_index("c")
    sid = lax.axis_index("s")
    wid = sid * NC + cid
    base = wid * QW
    qbase = wid * NQW

    iota = lax.iota(jnp.int32, 16)
    c1v = iota * 0 + 1
    c2v = iota * 0 + 2
    c3v = iota * 0 + 3

    ones16 = jnp.full((16,), 1.0, jnp.float32)
    for b in range(8):
        ones_v[pl.ds(b * 16, 16)] = ones16

    z16 = jnp.zeros((16,), jnp.float32)

    @pl.loop(0, SLC // 16)
    def _zero(i):
        zbuf_v[pl.ds(i * 16, 16)] = z16

    pltpu.sync_copy(zbuf_v, csh.at[pl.ds(sid * SLC, SLC)])

    PC = (pc0, pc1)
    QB = (qb0, qb1)
    PSEM = (psem0, psem1)
    QSEM = (qsem0, qsem1)

    def issue_pc(p, k):
        pltpu.async_copy(pc_hbm.at[pl.ds(qbase + k * 128, 128)], PC[p],
                         PSEM[p])

    def wait_pc(p):
        pltpu.make_async_copy(pc_hbm.at[pl.ds(0, 128)], PC[p],
                              PSEM[p]).wait()

    def issue_qst(p, k):
        pltpu.async_copy(QB[p], qtab_hbm.at[pl.ds(qbase + k * 128, 128)],
                         QSEM[p])

    def wait_qst(p):
        pltpu.make_async_copy(QB[p], qtab_hbm.at[pl.ds(0, 128)],
                              QSEM[p]).wait()

    def qcompute(p):
        for g in range(8):
            rv = g * 16 + iota
            xs = plsc.load_gather(PC[p], [rv, c1v])
            ys = plsc.load_gather(PC[p], [rv, c2v])
            zs = plsc.load_gather(PC[p], [rv, c3v])
            plsc.store_scatter(QB[p], [rv, c1v * 0], xs * (1.0 / VX))
            plsc.store_scatter(QB[p], [rv, c1v], ys * (1.0 / VY))
            plsc.store_scatter(QB[p], [rv, c2v], zs * (1.0 / VZ))

    issue_pc(0, 0)
    issue_pc(1, 1)

    @pl.loop(0, NQCH // 2)
    def _qt(t):
        for p in (0, 1):
            k = 2 * t + p
            wait_pc(p)

            @pl.when(k >= 2)
            def _():
                wait_qst(p)
            qcompute(p)
            issue_qst(p, k)

            @pl.when(k + 2 < NQCH)
            def _():
                issue_pc(p, k + 2)

    wait_qst(0)
    wait_qst(1)

    plsc.subcore_barrier()

    def issue_idx(idx_ref, sem, r):
        for b in range(8):
            pltpu.async_copy(
                vmap_hbm.at[pl.ds(base + (r * 8 + b) * 128, 128)],
                idx_ref.at[b], sem)

    def drain_idx(idx_ref, sem):
        for b in range(8):
            pltpu.make_async_copy(vmap_hbm.at[pl.ds(base, 128)],
                                  idx_ref.at[b], sem).wait()

    def issue_add(idx_ref, sem):
        for b in range(8):
            pltpu.async_copy(ones_v, csh.at[idx_ref.at[b]], sem, add=True)

    def drain_add(idx_ref, sem):
        for b in range(8):
            pltpu.make_async_copy(ones_v, csh.at[idx_ref.at[b]], sem).wait()

    issue_idx(idx_a, isem_a, 0)

    @pl.loop(0, RND_A // 2)
    def _t(t):
        @pl.when(t > 0)
        def _():
            drain_add(idx_b, asem_b)
        issue_idx(idx_b, isem_b, 2 * t + 1)
        drain_idx(idx_a, isem_a)
        issue_add(idx_a, asem_a)
        drain_add(idx_a, asem_a)

        @pl.when(t < RND_A // 2 - 1)
        def _():
            issue_idx(idx_a, isem_a, 2 * t + 2)
        drain_idx(idx_b, isem_b)
        issue_add(idx_b, asem_b)

    drain_add(idx_b, asem_b)
    plsc.subcore_barrier()
    pltpu.sync_copy(csh.at[pl.ds(sid * SLC, SLC)], zbuf_v)
    pltpu.sync_copy(zbuf_v, out_hbm.at[pl.ds(cid * NPAD + sid * SLC, SLC)])


@jax.jit
def _counts_call(vmap_pad, pc_pad):
    return pl.kernel(
        _counts_body,
        out_type=(jax.ShapeDtypeStruct((NC * NPAD,), jnp.float32),
                  jax.ShapeDtypeStruct((QTPAD, 16), jnp.float32)),
        mesh=_sc_mesh(),
        compiler_params=_SC_PARAMS,
        scratch_types=[
            pltpu.VMEM((8, 128), jnp.int32),
            pltpu.VMEM((8, 128), jnp.int32),
            pltpu.VMEM((128, 4), jnp.float32),
            pltpu.VMEM((128, 4), jnp.float32),
            pltpu.VMEM((128, 16), jnp.float32),
            pltpu.VMEM((128, 16), jnp.float32),
            pltpu.VMEM((128,), jnp.float32),
            pltpu.VMEM((SLC,), jnp.float32),
            pltpu.VMEM_SHARED((NPAD,), jnp.float32),
            pltpu.SemaphoreType.DMA,
            pltpu.SemaphoreType.DMA,
            pltpu.SemaphoreType.DMA,
            pltpu.SemaphoreType.DMA,
            pltpu.SemaphoreType.DMA,
            pltpu.SemaphoreType.DMA,
            pltpu.SemaphoreType.DMA,
            pltpu.SemaphoreType.DMA,
        ],
    )(vmap_pad, pc_pad)


BN = 2000
NB = N // BN


def _stats_body(cnt_ref, pf_ref, wt_ref, out_ref):
    i = pl.program_id(0)

    @pl.when(i == 0)
    def _():
        out_ref[...] = jnp.zeros_like(out_ref)

    h = jnp.dot(pf_ref[...], wt_ref[...], preferred_element_type=jnp.float32)
    cw = cnt_ref[0, 0, 0:1, :] + cnt_ref[1, 0, 0:1, :]
    out_ref[0:1, :] += jnp.dot(cw, h, preferred_element_type=jnp.float32)
    out_ref[1:2, :] += jnp.dot(cw, h * h,
                               preferred_element_type=jnp.float32)


@jax.jit
def _stats_call(counts4, p_features, wt):
    return pl.pallas_call(
        _stats_body,
        grid=(NB,),
        in_specs=[
            pl.BlockSpec((2, 1, 1, BN), lambda i: (0, i, 0, 0)),
            pl.BlockSpec((BN, C), lambda i: (i, 0)),
            pl.BlockSpec((C, C), lambda i: (0, 0)),
        ],
        out_specs=pl.BlockSpec((2, C), lambda i: (0, 0)),
        out_shape=jax.ShapeDtypeStruct((2, C), jnp.float32),
    )(counts4, p_features, wt)


def _apply_body(pf_ref, wt_ref, ab_ref, out_ref):
    h = jnp.dot(pf_ref[...], wt_ref[...], preferred_element_type=jnp.float32)
    out_ref[...] = jnp.maximum(h * ab_ref[0:1, :] + ab_ref[1:2, :], 0.0)


@jax.jit
def _apply_call(p_features, wt, ab):
    return pl.pallas_call(
        _apply_body,
        grid=(NB,),
        in_specs=[
            pl.BlockSpec((BN, C), lambda i: (i, 0)),
            pl.BlockSpec((C, C), lambda i: (0, 0)),
            pl.BlockSpec((2, C), lambda i: (0, 0)),
        ],
        out_specs=pl.BlockSpec((BN, C), lambda i: (i, 0)),
        out_shape=jax.ShapeDtypeStruct((N, C), jnp.float32),
    )(p_features, wt, ab)


def _pool_body(hpp_hbm, qtab_hbm, vmapf_hbm, vi_hbm, vm_hbm, vf_hbm, out_hbm,
               idx0, idx1, rows0, rows1, qrows0, qrows1, vib0, vib1,
               vmb0, vmb1, vfb0, vfb1, outb0, outb1, wbuf,
               isem0, isem1, gsem0, gsem1, qsem0, qsem1, ssem0, ssem1,
               osem0, osem1):
    cid = lax.axis_index("c")
    sid = lax.axis_index("s")
    wid = sid * NC + cid
    cbase = wid * WCH

    IDX = (idx0, idx1)
    ROWS = (rows0, rows1)
    QROWS = (qrows0, qrows1)
    VIB = (vib0, vib1)
    VMB = (vmb0, vmb1)
    VFB = (vfb0, vfb1)
    OUTB = (outb0, outb1)
    ISEM = (isem0, isem1)
    GSEM = (gsem0, gsem1)
    QSEM = (qsem0, qsem1)
    SSEM = (ssem0, ssem1)
    OSEM = (osem0, osem1)

    iota = lax.iota(jnp.int32, 16)
    c0v = iota * 0
    c1v = c0v + 1
    c2v = c0v + 2
    c3v = c0v + 3
    RIDX = [i * NS + iota for i in range(CH)]
    FULLI = [c0v + i for i in range(CH)]
    WIDX = [(iota % 4) * 16 + s for s in range(NS)]

    def vbase(k):
        return (cbase + k) * CH

    def issue_idx(p, k):
        pltpu.async_copy(vmapf_hbm.at[pl.ds(vbase(k) * NS, ROWSN)],
                         IDX[p], ISEM[p])

    def wait_idx(p):
        pltpu.make_async_copy(vmapf_hbm.at[pl.ds(0, ROWSN)],
                              IDX[p], ISEM[p]).wait()

    def issue_main(p, k):
        pltpu.async_copy(hpp_hbm.at[IDX[p]], ROWS[p], GSEM[p])
        pltpu.async_copy(qtab_hbm.at[IDX[p]], QROWS[p], QSEM[p])
        vb = vbase(k)
        pltpu.async_copy(vi_hbm.at[pl.ds(vb, CH)], VIB[p], SSEM[p])
        pltpu.async_copy(vm_hbm.at[pl.ds(vb, CH)], VMB[p], SSEM[p])
        pltpu.async_copy(vf_hbm.at[pl.ds(vb, CH)], VFB[p], SSEM[p])

    def wait_main(p):
        pltpu.make_async_copy(hpp_hbm.at[IDX[p]], ROWS[p], GSEM[p]).wait()
        pltpu.make_async_copy(qtab_hbm.at[IDX[p]], QROWS[p], QSEM[p]).wait()
        pltpu.make_async_copy(vi_hbm.at[pl.ds(0, CH)], VIB[p],
                              SSEM[p]).wait()
        pltpu.make_async_copy(vm_hbm.at[pl.ds(0, CH)], VMB[p],
                              SSEM[p]).wait()
        pltpu.make_async_copy(vf_hbm.at[pl.ds(0, CH)], VFB[p],
                              SSEM[p]).wait()

    def issue_out(p, k):
        pltpu.async_copy(OUTB[p], out_hbm.at[pl.ds(vbase(k), CH)], OSEM[p])

    def wait_out(p):
        pltpu.make_async_copy(OUTB[p], out_hbm.at[pl.ds(0, CH)],
                              OSEM[p]).wait()

    def compute(p):
        rows_v = ROWS[p]
        qrows_v = QROWS[p]
        vi_v = VIB[p]
        vm_v = VMB[p]
        vf_v = VFB[p]
        out_v = OUTB[p]
        for i in range(CH):
            qx = plsc.load_gather(qrows_v, [RIDX[i], c0v])
            qy = plsc.load_gather(qrows_v, [RIDX[i], c1v])
            qz = plsc.load_gather(qrows_v, [RIDX[i], c2v])
            xi = plsc.load_gather(vi_v, [FULLI[i], c3v])
            yi = plsc.load_gather(vi_v, [FULLI[i], c2v])
            zi = plsc.load_gather(vi_v, [FULLI[i], c1v])
            ux = xi.astype(jnp.float32) + CX0
            uy = yi.astype(jnp.float32) + CY0
            uz = zi.astype(jnp.float32) + CZ0
            vm_row = vm_v[i, :]
            den = jnp.maximum(jnp.zeros((16,), jnp.float32)
                              + jnp.sum(vm_row), 1.0)
            mds = vm_row / den
            g0 = qx - ux
            g1 = qy - uy
            g2 = qz - uz
            g3 = S0 * (g0 * g0) + S1 * (g1 * g1) + S2 * (g2 * g2)
            wbuf[pl.ds(0, 16)] = g0 * mds
            wbuf[pl.ds(16, 16)] = g1 * mds
            wbuf[pl.ds(32, 16)] = g2 * mds
            wbuf[pl.ds(48, 16)] = g3 * mds
            acc = [vf_v[i, pl.ds(j * 16, 16)] for j in range(8)]
            for s in range(NS):
                wv = plsc.load_gather(wbuf, [WIDX[s]])
                r = i * NS + s
                for j in range(8):
                    acc[j] = acc[j] + wv * rows_v[r, pl.ds(j * 16, 16)]
            for j in range(8):
                out_v[i, pl.ds(j * 16, 16)] = acc[j]

    issue_idx(0, 0)
    wait_idx(0)
    issue_main(0, 0)
    issue_idx(1, 1)

    @pl.loop(0, WCH // 2)
    def _t(t):
        for p in (0, 1):
            k = 2 * t + p

            @pl.when(k + 1 < WCH)
            def _():
                wait_idx(1 - p)
                issue_main(1 - p, k + 1)
            wait_main(p)

            @pl.when(k + 2 < WCH)
            def _():
                issue_idx(p, k + 2)

            @pl.when(k >= 2)
            def _():
                wait_out(p)
            compute(p)
            issue_out(p, k)

    wait_out(0)
    wait_out(1)


@jax.jit
def _pool_call(hpp, qtab, vmapf, vi_pad, vm_pad, vf_pad):
    return pl.kernel(
        _pool_body,
        out_type=jax.ShapeDtypeStruct((MPAD, C), jnp.float32),
        mesh=_sc_mesh(),
        compiler_params=_SC_PARAMS,
        scratch_types=[
            pltpu.VMEM((ROWSN,), jnp.int32),
            pltpu.VMEM((ROWSN,), jnp.int32),
            pltpu.VMEM((ROWSN, C), jnp.float32),
            pltpu.VMEM((ROWSN, C), jnp.float32),
            pltpu.VMEM((ROWSN, 16), jnp.float32),
            pltpu.VMEM((ROWSN, 16), jnp.float32),
            pltpu.VMEM((CH, 4), jnp.int32),
            pltpu.VMEM((CH, 4), jnp.int32),
            pltpu.VMEM((CH, 16), jnp.float32),
            pltpu.VMEM((CH, 16), jnp.float32),
            pltpu.VMEM((CH, C), jnp.float32),
            pltpu.VMEM((CH, C), jnp.float32),
            pltpu.VMEM((CH, C), jnp.float32),
            pltpu.VMEM((CH, C), jnp.float32),
            pltpu.VMEM((64,), jnp.float32),
            pltpu.SemaphoreType.DMA,
            pltpu.SemaphoreType.DMA,
            pltpu.SemaphoreType.DMA,
            pltpu.SemaphoreType.DMA,
            pltpu.SemaphoreType.DMA,
            pltpu.SemaphoreType.DMA,
            pltpu.SemaphoreType.DMA,
            pltpu.SemaphoreType.DMA,
            pltpu.SemaphoreType.DMA,
            pltpu.SemaphoreType.DMA,
        ],
    )(hpp, qtab, vmapf, vi_pad, vm_pad, vf_pad)


def kernel(p_coords, p_features, v_indices, v_features, v_map, v_mask, W,
           gamma, beta):
    vmap_flat = v_map.astype(jnp.int32).reshape(-1)
    vmap_a = jnp.concatenate(
        [vmap_flat, jnp.full((VTOT_PAD - VTOT,), N, jnp.int32)])
    pc_pad = jnp.pad(p_coords, ((0, QTPAD - N), (0, 0)))
    counts_flat, qtab = _counts_call(vmap_a, pc_pad)
    counts4 = jnp.stack([counts_flat[:N].reshape(NB, 1, BN),
                         counts_flat[NPAD:NPAD + N].reshape(NB, 1, BN)])

    wt = W.T
    stats = _stats_call(counts4, p_features, wt)
    rtot = float(M * NS)
    mean = stats[0] / rtot
    var = stats[1] / rtot - mean * mean
    a = gamma * lax.rsqrt(var + 1e-5)
    b = beta - mean * a
    ab = jnp.stack([a, b])

    hpp = _apply_call(p_features, wt, ab)

    vi_pad = jnp.pad(v_indices.astype(jnp.int32), ((0, MPAD - M), (0, 0)))
    vm_pad = jnp.pad(v_mask, ((0, MPAD - M), (0, 0)))
    vf_pad = jnp.pad(v_features, ((0, MPAD - M), (0, 0)))
    vmap_c = jnp.pad(vmap_flat, (0, (MPAD - M) * NS))

    out = _pool_call(hpp, qtab, vmap_c, vi_pad, vm_pad, vf_pad)
    return out[:M]
